# baseline (speedup 1.0000x reference)
import jax
import jax.numpy as jnp
from jax.experimental import pallas as pl
from jax.experimental.pallas import tpu as pltpu


def _fused_kernel(x_ref, w_ref, b_ref, adj_ref, o_ref, sup_ref):
    @pl.when(pl.program_id(1) == 0)
    def _():
        sup_ref[...] = jnp.dot(
            x_ref[...].astype(jnp.bfloat16),
            w_ref[...].astype(jnp.bfloat16),
            preferred_element_type=jnp.float32,
        ).astype(jnp.bfloat16)

    a = adj_ref[...].astype(jnp.bfloat16)
    o_ref[...] = (
        jnp.dot(a, sup_ref[...], preferred_element_type=jnp.float32)
        + b_ref[...]
    )


def kernel(input_features, adj, weight, bias):
    n, f_in = input_features.shape
    f_out = weight.shape[1]

    bias_p = bias.reshape(1, f_out).astype(jnp.float32)

    n_cores = 2 if n % 1024 == 0 else 1
    tm = min(256, n)
    n_j = n // (tm * n_cores)

    out = pl.pallas_call(
        _fused_kernel,
        out_shape=jax.ShapeDtypeStruct((n, f_out), jnp.float32),
        grid=(n_cores, n_j),
        in_specs=[
            pl.BlockSpec((n, f_in), lambda i, j: (0, 0)),
            pl.BlockSpec((f_in, f_out), lambda i, j: (0, 0)),
            pl.BlockSpec((1, f_out), lambda i, j: (0, 0)),
            pl.BlockSpec((tm, n), lambda i, j: (i * n_j + j, 0)),
        ],
        out_specs=pl.BlockSpec((tm, f_out), lambda i, j: (i * n_j + j, 0)),
        scratch_shapes=[pltpu.VMEM((n, f_out), jnp.bfloat16)],
        compiler_params=pltpu.CompilerParams(
            dimension_semantics=("parallel", "arbitrary"),
            vmem_limit_bytes=int(56 << 20),
        ),
    )(input_features, weight, bias_p, adj)

    return out


# speedup vs baseline: 1.1309x; 1.1309x over previous
import jax
import jax.numpy as jnp
from jax.experimental import pallas as pl
from jax.experimental.pallas import tpu as pltpu


def _fused_kernel(x_ref, w_ref, b_ref, adj_ref, o_ref, sup_ref):
    @pl.when(pl.program_id(1) == 0)
    def _():
        sup_ref[...] = jnp.dot(
            x_ref[...].astype(jnp.bfloat16),
            w_ref[...].astype(jnp.bfloat16),
            preferred_element_type=jnp.float32,
        ).astype(jnp.bfloat16)

    a = adj_ref[...].astype(jnp.bfloat16)
    o_ref[...] = (
        jnp.dot(a, sup_ref[...], preferred_element_type=jnp.float32)
        + b_ref[...]
    )


def kernel(input_features, adj, weight, bias):
    n, f_in = input_features.shape
    f_out = weight.shape[1]

    bias_p = bias.reshape(1, f_out).astype(jnp.float32)

    n_cores = 2 if n % 1024 == 0 else 1
    tm = min(1024, n)
    n_j = n // (tm * n_cores)

    out = pl.pallas_call(
        _fused_kernel,
        out_shape=jax.ShapeDtypeStruct((n, f_out), jnp.float32),
        grid=(n_cores, n_j),
        in_specs=[
            pl.BlockSpec((n, f_in), lambda i, j: (0, 0)),
            pl.BlockSpec((f_in, f_out), lambda i, j: (0, 0)),
            pl.BlockSpec((1, f_out), lambda i, j: (0, 0)),
            pl.BlockSpec((tm, n), lambda i, j: (i * n_j + j, 0)),
        ],
        out_specs=pl.BlockSpec((tm, f_out), lambda i, j: (i * n_j + j, 0)),
        scratch_shapes=[pltpu.VMEM((n, f_out), jnp.bfloat16)],
        compiler_params=pltpu.CompilerParams(
            dimension_semantics=("parallel", "arbitrary"),
            vmem_limit_bytes=int(56 << 20),
        ),
    )(input_features, weight, bias_p, adj)

    return out


# speedup vs baseline: 1.1503x; 1.0172x over previous
import jax
import jax.numpy as jnp
from jax.experimental import pallas as pl
from jax.experimental.pallas import tpu as pltpu


def _fused_kernel(x_ref, w_ref, b_ref, adj_ref, o_ref, sup_ref):
    @pl.when(pl.program_id(1) == 0)
    def _():
        sup_ref[...] = jnp.dot(
            x_ref[...].astype(jnp.bfloat16),
            w_ref[...].astype(jnp.bfloat16),
            preferred_element_type=jnp.float32,
        ).astype(jnp.bfloat16)

    a = adj_ref[...].astype(jnp.bfloat16)
    o_ref[...] = (
        jnp.dot(a, sup_ref[...], preferred_element_type=jnp.float32)
        + b_ref[...]
    )


def kernel(input_features, adj, weight, bias):
    n, f_in = input_features.shape
    f_out = weight.shape[1]

    bias_p = bias.reshape(1, f_out).astype(jnp.float32)

    n_cores = 2 if n % 1024 == 0 else 1
    tm = min(512, n)
    n_j = n // (tm * n_cores)

    out = pl.pallas_call(
        _fused_kernel,
        out_shape=jax.ShapeDtypeStruct((n, f_out), jnp.float32),
        grid=(n_cores, n_j),
        in_specs=[
            pl.BlockSpec((n, f_in), lambda i, j: (0, 0)),
            pl.BlockSpec((f_in, f_out), lambda i, j: (0, 0)),
            pl.BlockSpec((1, f_out), lambda i, j: (0, 0)),
            pl.BlockSpec((tm, n), lambda i, j: (i * n_j + j, 0)),
        ],
        out_specs=pl.BlockSpec((tm, f_out), lambda i, j: (i * n_j + j, 0)),
        scratch_shapes=[pltpu.VMEM((n, f_out), jnp.bfloat16)],
        compiler_params=pltpu.CompilerParams(
            dimension_semantics=("parallel", "arbitrary"),
            vmem_limit_bytes=int(56 << 20),
        ),
    )(input_features, weight, bias_p, adj)

    return out


# speedup vs baseline: 1.1516x; 1.0011x over previous
import jax
import jax.numpy as jnp
from jax.experimental import pallas as pl
from jax.experimental.pallas import tpu as pltpu


def _fused_kernel(adj_ref, x_ref, w_ref, b_ref, o_ref, sup_ref):
    @pl.when(pl.program_id(1) == 0)
    def _():
        sup_ref[...] = jnp.dot(
            x_ref[...].astype(jnp.bfloat16),
            w_ref[...].astype(jnp.bfloat16),
            preferred_element_type=jnp.float32,
        ).astype(jnp.bfloat16)

    a = adj_ref[...].astype(jnp.bfloat16)
    o_ref[...] = (
        jnp.dot(a, sup_ref[...], preferred_element_type=jnp.float32)
        + b_ref[...]
    )


def kernel(input_features, adj, weight, bias):
    n, f_in = input_features.shape
    f_out = weight.shape[1]

    bias_p = bias.reshape(1, f_out).astype(jnp.float32)

    n_cores = 2 if n % 1024 == 0 else 1
    tm = min(512, n)
    n_j = n // (tm * n_cores)

    out = pl.pallas_call(
        _fused_kernel,
        out_shape=jax.ShapeDtypeStruct((n, f_out), jnp.float32),
        grid=(n_cores, n_j),
        in_specs=[
            pl.BlockSpec((tm, n), lambda i, j: (i * n_j + j, 0)),
            pl.BlockSpec((n, f_in), lambda i, j: (0, 0)),
            pl.BlockSpec((f_in, f_out), lambda i, j: (0, 0)),
            pl.BlockSpec((1, f_out), lambda i, j: (0, 0)),
        ],
        out_specs=pl.BlockSpec((tm, f_out), lambda i, j: (i * n_j + j, 0)),
        scratch_shapes=[pltpu.VMEM((n, f_out), jnp.bfloat16)],
        compiler_params=pltpu.CompilerParams(
            dimension_semantics=("parallel", "arbitrary"),
            vmem_limit_bytes=int(56 << 20),
        ),
    )(adj, input_features, weight, bias_p)

    return out


# speedup vs baseline: 1.1622x; 1.0092x over previous
import jax
import jax.numpy as jnp
from jax.experimental import pallas as pl
from jax.experimental.pallas import tpu as pltpu


def _fused_kernel(adj_ref, x_ref, w_ref, b_ref, o_ref, sup_ref):
    @pl.when(pl.program_id(1) == 0)
    def _():
        sup_ref[...] = jnp.dot(
            x_ref[...].astype(jnp.bfloat16),
            w_ref[...].astype(jnp.bfloat16),
            preferred_element_type=jnp.float32,
        ).astype(jnp.bfloat16)

    a = adj_ref[...].astype(jnp.bfloat16)
    tm = a.shape[0]
    o_ref[pl.ds(pl.program_id(1) * tm, tm), :] = (
        jnp.dot(a, sup_ref[...], preferred_element_type=jnp.float32)
        + b_ref[...]
    )


def kernel(input_features, adj, weight, bias):
    n, f_in = input_features.shape
    f_out = weight.shape[1]

    bias_p = bias.reshape(1, f_out).astype(jnp.float32)

    n_cores = 2 if n % 1024 == 0 else 1
    tm = min(512, n)
    n_j = n // (tm * n_cores)

    out = pl.pallas_call(
        _fused_kernel,
        out_shape=jax.ShapeDtypeStruct((n, f_out), jnp.float32),
        grid=(n_cores, n_j),
        in_specs=[
            pl.BlockSpec((tm, n), lambda i, j: (i * n_j + j, 0)),
            pl.BlockSpec((n, f_in), lambda i, j: (0, 0)),
            pl.BlockSpec((f_in, f_out), lambda i, j: (0, 0)),
            pl.BlockSpec((1, f_out), lambda i, j: (0, 0)),
        ],
        out_specs=pl.BlockSpec((tm * n_j, f_out), lambda i, j: (i, 0)),
        scratch_shapes=[pltpu.VMEM((n, f_out), jnp.bfloat16)],
        compiler_params=pltpu.CompilerParams(
            dimension_semantics=("parallel", "arbitrary"),
            vmem_limit_bytes=int(56 << 20),
        ),
    )(adj, input_features, weight, bias_p)

    return out
